# baseline (speedup 1.0000x reference)
import functools

import jax
import jax.numpy as jnp
from jax import lax
from jax.experimental import pallas as pl
from jax.experimental.pallas import tpu as pltpu
from jax.experimental.pallas import tpu_sc as plsc

POOL_SIZE = 30
PROMPT_LEN = 5
TOP_K = 5
EMBED_DIM = 768
BATCH = 128
TOKENS = 196
TOK_BLK = 14

_NSLAB = TOP_K * PROMPT_LEN
_NSLAB_PAD = 32


def _topk_body(x_ref, p_ref, idx_ref, maxs, sums):
    i = pl.program_id(0)
    n = pl.num_programs(0)
    x = x_ref[...]
    bmax = jnp.max(x, axis=0)
    bsum = jnp.sum(x, axis=0)

    @pl.when(i == 0)
    def _():
        maxs[...] = bmax
        sums[...] = bsum

    @pl.when(i > 0)
    def _():
        maxs[...] = jnp.maximum(maxs[...], bmax)
        sums[...] = sums[...] + bsum

    @pl.when(i == n - 1)
    def _():
        embed_keys = maxs[...] + 2.0 * (sums[...] * (1.0 / TOKENS))
        keys = jnp.sum(p_ref[...], axis=0) * (1.0 / PROMPT_LEN)

        def l2(v):
            ss = jnp.sum(v * v, axis=1, keepdims=True)
            return v * lax.rsqrt(jnp.maximum(ss, 1e-12))

        sim = lax.dot_general(
            l2(embed_keys), l2(keys),
            dimension_numbers=(((1,), (1,)), ((), ())),
            preferred_element_type=jnp.float32)

        col = lax.broadcasted_iota(jnp.int32, sim.shape, 1)
        rows = []
        for _ in range(TOP_K):
            m = jnp.max(sim, axis=1, keepdims=True)
            idx = jnp.min(jnp.where(sim == m, col, POOL_SIZE), axis=1)
            rows.extend((idx + t * POOL_SIZE)[None, :] for t in range(PROMPT_LEN))
            sim = jnp.where(col == idx[:, None], -jnp.inf, sim)
        rows.append(jnp.zeros((_NSLAB_PAD - _NSLAB, BATCH), jnp.int32))
        idx_ref[...] = jnp.concatenate(rows, axis=0)


def _topk_idx(x_t, p_t):
    return pl.pallas_call(
        _topk_body,
        grid=(TOKENS // TOK_BLK,),
        in_specs=[
            pl.BlockSpec((TOK_BLK, BATCH, EMBED_DIM), lambda i: (i, 0, 0)),
            pl.BlockSpec((PROMPT_LEN, POOL_SIZE, EMBED_DIM), lambda i: (0, 0, 0)),
        ],
        out_specs=pl.BlockSpec((_NSLAB_PAD, BATCH), lambda i: (0, 0)),
        out_shape=jax.ShapeDtypeStruct((_NSLAB_PAD, BATCH), jnp.int32),
        scratch_shapes=[
            pltpu.VMEM((BATCH, EMBED_DIM), jnp.float32),
            pltpu.VMEM((BATCH, EMBED_DIM), jnp.float32),
        ],
    )(x_t, p_t)


def _sc_gather(table, idx):
    mesh = plsc.VectorSubcoreMesh(core_axis_name="c", subcore_axis_name="s")

    @functools.partial(
        pl.kernel,
        mesh=mesh,
        out_type=jax.ShapeDtypeStruct((_NSLAB, BATCH, EMBED_DIM), jnp.float32),
        scratch_types=[
            pltpu.VMEM((BATCH,), jnp.int32),
            pltpu.VMEM((BATCH, EMBED_DIM), jnp.float32),
            pltpu.SemaphoreType.DMA,
        ],
    )
    def k(table_hbm, idx_hbm, out_hbm, idx_v, rows_v, sem):
        wid = lax.axis_index("s") * 2 + lax.axis_index("c")

        @pl.when(wid < _NSLAB)
        def _():
            pltpu.sync_copy(idx_hbm.at[wid], idx_v)
            pltpu.async_copy(table_hbm.at[idx_v], rows_v, sem).wait()
            pltpu.sync_copy(rows_v, out_hbm.at[wid])

    return k(table, idx)


def kernel(input_embed, prompt):
    x_t = jnp.transpose(input_embed, (1, 0, 2))
    p_t = jnp.transpose(prompt, (1, 0, 2))
    idx = _topk_idx(x_t, p_t)
    table = p_t.reshape(PROMPT_LEN * POOL_SIZE, EMBED_DIM)
    out_t = _sc_gather(table, idx)
    return jnp.transpose(out_t, (1, 0, 2))


# speedup vs baseline: 1.0005x; 1.0005x over previous
import functools

import jax
import jax.numpy as jnp
from jax import lax
from jax.experimental import pallas as pl
from jax.experimental.pallas import tpu as pltpu
from jax.experimental.pallas import tpu_sc as plsc

POOL_SIZE = 30
PROMPT_LEN = 5
TOP_K = 5
EMBED_DIM = 768
BATCH = 128
TOKENS = 196
TOK_BLK = 28

_NSLAB = TOP_K * PROMPT_LEN
_NSLAB_PAD = 32
_NSLAB_TC = 15
_NSLAB_SC = _NSLAB - _NSLAB_TC
_HALF = BATCH // 2


def _topk_body(x_ref, p_ref, idx_ref, idxb_ref, maxs, sums):
    i = pl.program_id(0)
    n = pl.num_programs(0)
    x = x_ref[...]
    bmax = jnp.max(x, axis=0)
    bsum = jnp.sum(x, axis=0)

    @pl.when(i == 0)
    def _():
        maxs[...] = bmax
        sums[...] = bsum

    @pl.when(i > 0)
    def _():
        maxs[...] = jnp.maximum(maxs[...], bmax)
        sums[...] = sums[...] + bsum

    @pl.when(i == n - 1)
    def _():
        embed_keys = maxs[...] + 2.0 * (sums[...] * (1.0 / TOKENS))
        keys = jnp.sum(p_ref[...], axis=0) * (1.0 / PROMPT_LEN)

        def l2(v):
            ss = jnp.sum(v * v, axis=1, keepdims=True)
            return v * lax.rsqrt(jnp.maximum(ss, 1e-12))

        sim = lax.dot_general(
            l2(embed_keys), l2(keys),
            dimension_numbers=(((1,), (1,)), ((), ())),
            preferred_element_type=jnp.float32)

        col = lax.broadcasted_iota(jnp.int32, sim.shape, 1)
        rows, cols = [], []
        for _ in range(TOP_K):
            m = jnp.max(sim, axis=1, keepdims=True)
            idx = jnp.min(jnp.where(sim == m, col, POOL_SIZE), axis=1)
            rows.extend((idx + t * POOL_SIZE)[None, :] for t in range(PROMPT_LEN))
            cols.extend((idx + t * POOL_SIZE)[:, None] for t in range(PROMPT_LEN))
            sim = jnp.where(col == idx[:, None], -jnp.inf, sim)
        rows.append(jnp.zeros((_NSLAB_PAD - _NSLAB, BATCH), jnp.int32))
        cols.append(jnp.zeros((BATCH, _NSLAB_PAD - _NSLAB), jnp.int32))
        idx_ref[...] = jnp.concatenate(rows, axis=0)
        idxb_ref[...] = jnp.concatenate(cols, axis=1)


def _topk_idx(x_t, p_t):
    return pl.pallas_call(
        _topk_body,
        grid=(TOKENS // TOK_BLK,),
        in_specs=[
            pl.BlockSpec((TOK_BLK, BATCH, EMBED_DIM), lambda i: (i, 0, 0)),
            pl.BlockSpec((PROMPT_LEN, POOL_SIZE, EMBED_DIM), lambda i: (0, 0, 0)),
        ],
        out_specs=[
            pl.BlockSpec((_NSLAB_PAD, BATCH), lambda i: (0, 0)),
            pl.BlockSpec((BATCH, _NSLAB_PAD), lambda i: (0, 0)),
        ],
        out_shape=[
            jax.ShapeDtypeStruct((_NSLAB_PAD, BATCH), jnp.int32),
            jax.ShapeDtypeStruct((BATCH, _NSLAB_PAD), jnp.int32),
        ],
        scratch_shapes=[
            pltpu.VMEM((BATCH, EMBED_DIM), jnp.float32),
            pltpu.VMEM((BATCH, EMBED_DIM), jnp.float32),
        ],
    )(x_t, p_t)


def _sc_gather(table, idx):
    mesh = plsc.VectorSubcoreMesh(core_axis_name="c", subcore_axis_name="s")

    @functools.partial(
        pl.kernel,
        mesh=mesh,
        out_type=jax.ShapeDtypeStruct((_NSLAB, BATCH, EMBED_DIM), jnp.float32),
        scratch_types=[
            pltpu.VMEM((BATCH,), jnp.int32),
            pltpu.VMEM((_HALF, EMBED_DIM), jnp.float32),
            pltpu.SemaphoreType.DMA,
        ],
    )
    def k(table_hbm, idx_hbm, out_hbm, idx_v, rows_v, sem):
        wid = lax.axis_index("s") * 2 + lax.axis_index("c")

        @pl.when(wid < 2 * _NSLAB_SC)
        def _():
            slab = _NSLAB_TC + lax.div(wid, 2)
            half = lax.rem(wid, 2)
            pltpu.sync_copy(idx_hbm.at[slab], idx_v)
            pltpu.async_copy(
                table_hbm.at[idx_v.at[pl.ds(half * _HALF, _HALF)]],
                rows_v, sem).wait()
            pltpu.sync_copy(rows_v, out_hbm.at[slab, pl.ds(half * _HALF, _HALF)])

    return k(table, idx)


def _tc_scatter_body(out_any, p_ref, idxb_ref, out_ref):
    j = pl.program_id(0)
    t = lax.rem(j, PROMPT_LEN)
    idxm = idxb_ref[...]
    lane = lax.broadcasted_iota(jnp.int32, idxm.shape, 1)
    ids = jnp.sum(jnp.where(lane == j, idxm, 0), axis=1,
                  keepdims=True) - t * POOL_SIZE
    sel = (lax.broadcasted_iota(jnp.int32, (BATCH, POOL_SIZE), 1)
           == ids).astype(jnp.float32)
    slab = lax.dot_general(
        sel, p_ref[0],
        dimension_numbers=(((1,), (0,)), ((), ())),
        preferred_element_type=jnp.float32)
    out_ref[...] = slab[None]


def _tc_scatter(out_sc, p_t, idxb):
    return pl.pallas_call(
        _tc_scatter_body,
        grid=(_NSLAB_TC,),
        in_specs=[
            pl.BlockSpec(memory_space=pl.ANY),
            pl.BlockSpec((1, POOL_SIZE, EMBED_DIM),
                         lambda j: (j % PROMPT_LEN, 0, 0)),
            pl.BlockSpec((BATCH, _NSLAB_PAD), lambda j: (0, 0)),
        ],
        out_specs=pl.BlockSpec((1, BATCH, EMBED_DIM), lambda j: (j, 0, 0)),
        out_shape=jax.ShapeDtypeStruct((_NSLAB, BATCH, EMBED_DIM),
                                       jnp.float32),
        input_output_aliases={0: 0},
    )(out_sc, p_t, idxb)


def kernel(input_embed, prompt):
    x_t = jnp.transpose(input_embed, (1, 0, 2))
    p_t = jnp.transpose(prompt, (1, 0, 2))
    idx, idxb = _topk_idx(x_t, p_t)
    table = p_t.reshape(PROMPT_LEN * POOL_SIZE, EMBED_DIM)
    out_sc = _sc_gather(table, idx)
    out_t = _tc_scatter(out_sc, p_t, idxb)
    return jnp.transpose(out_t, (1, 0, 2))


# speedup vs baseline: 1.1142x; 1.1137x over previous
import functools

import jax
import jax.numpy as jnp
from jax import lax
from jax.experimental import pallas as pl
from jax.experimental.pallas import tpu as pltpu
from jax.experimental.pallas import tpu_sc as plsc

POOL_SIZE = 30
PROMPT_LEN = 5
TOP_K = 5
EMBED_DIM = 768
BATCH = 128
TOKENS = 196
TOK_BLK = 28

_NSLAB = TOP_K * PROMPT_LEN
_NSLAB_PAD = 32
_NSLAB_TC = 15
_NSLAB_SC = _NSLAB - _NSLAB_TC
_HALF = BATCH // 2


def _topk_body(x_ref, p_ref, idx_ref, idxb_ref, maxs, sums):
    i = pl.program_id(0)
    n = pl.num_programs(0)
    x = x_ref[...]
    bmax = jnp.max(x, axis=0)
    bsum = jnp.sum(x, axis=0)

    @pl.when(i == 0)
    def _():
        maxs[...] = bmax
        sums[...] = bsum

    @pl.when(i > 0)
    def _():
        maxs[...] = jnp.maximum(maxs[...], bmax)
        sums[...] = sums[...] + bsum

    @pl.when(i == n - 1)
    def _():
        embed_keys = maxs[...] + 2.0 * (sums[...] * (1.0 / TOKENS))
        keys = jnp.sum(p_ref[...], axis=0) * (1.0 / PROMPT_LEN)

        def l2(v):
            ss = jnp.sum(v * v, axis=1, keepdims=True)
            return v * lax.rsqrt(jnp.maximum(ss, 1e-12))

        sim = lax.dot_general(
            l2(embed_keys), l2(keys),
            dimension_numbers=(((1,), (1,)), ((), ())),
            preferred_element_type=jnp.float32)

        col = lax.broadcasted_iota(jnp.int32, sim.shape, 1)
        rows, cols = [], []
        for _ in range(TOP_K):
            m = jnp.max(sim, axis=1, keepdims=True)
            idx = jnp.min(jnp.where(sim == m, col, POOL_SIZE), axis=1)
            rows.extend((idx + t * POOL_SIZE)[None, :] for t in range(PROMPT_LEN))
            cols.extend((idx + t * POOL_SIZE)[:, None] for t in range(PROMPT_LEN))
            sim = jnp.where(col == idx[:, None], -jnp.inf, sim)
        rows.append(jnp.zeros((_NSLAB_PAD - _NSLAB, BATCH), jnp.int32))
        cols.append(jnp.zeros((BATCH, _NSLAB_PAD - _NSLAB), jnp.int32))
        idx_ref[...] = jnp.concatenate(rows, axis=0)
        idxb_ref[...] = jnp.concatenate(cols, axis=1)


def _topk_idx(x_t, p_t):
    return pl.pallas_call(
        _topk_body,
        grid=(TOKENS // TOK_BLK,),
        in_specs=[
            pl.BlockSpec((TOK_BLK, BATCH, EMBED_DIM), lambda i: (i, 0, 0)),
            pl.BlockSpec((PROMPT_LEN, POOL_SIZE, EMBED_DIM), lambda i: (0, 0, 0)),
        ],
        out_specs=[
            pl.BlockSpec((_NSLAB_PAD, BATCH), lambda i: (0, 0)),
            pl.BlockSpec((BATCH, _NSLAB_PAD), lambda i: (0, 0)),
        ],
        out_shape=[
            jax.ShapeDtypeStruct((_NSLAB_PAD, BATCH), jnp.int32),
            jax.ShapeDtypeStruct((BATCH, _NSLAB_PAD), jnp.int32),
        ],
        scratch_shapes=[
            pltpu.VMEM((BATCH, EMBED_DIM), jnp.float32),
            pltpu.VMEM((BATCH, EMBED_DIM), jnp.float32),
        ],
    )(x_t, p_t)


def _sc_gather(table, idx):
    mesh = plsc.VectorSubcoreMesh(core_axis_name="c", subcore_axis_name="s")

    @functools.partial(
        pl.kernel,
        mesh=mesh,
        out_type=jax.ShapeDtypeStruct((_NSLAB, BATCH, EMBED_DIM), jnp.float32),
        scratch_types=[
            pltpu.VMEM((BATCH,), jnp.int32),
            pltpu.VMEM((_HALF, EMBED_DIM), jnp.float32),
            pltpu.SemaphoreType.DMA,
        ],
    )
    def k(table_hbm, idx_hbm, out_hbm, idx_v, rows_v, sem):
        wid = lax.axis_index("s") * 2 + lax.axis_index("c")

        @pl.when(wid < 2 * _NSLAB_SC)
        def _():
            slab = _NSLAB_TC + lax.div(wid, 2)
            half = lax.rem(wid, 2)
            pltpu.sync_copy(idx_hbm.at[slab], idx_v)
            pltpu.async_copy(
                table_hbm.at[idx_v.at[pl.ds(half * _HALF, _HALF)]],
                rows_v, sem).wait()
            pltpu.sync_copy(rows_v, out_hbm.at[slab, pl.ds(half * _HALF, _HALF)])

    return k(table, idx)


def _tc_scatter_body(out_any, p_ref, idxb_ref, out_ref):
    s = pl.program_id(0)
    p = p_ref[...]
    idxm = idxb_ref[...]
    lane = lax.broadcasted_iota(jnp.int32, idxm.shape, 1)
    pool_iota = lax.broadcasted_iota(jnp.int32, (BATCH, POOL_SIZE), 1)
    slabs = []
    for t in range(PROMPT_LEN):
        j = s * PROMPT_LEN + t
        ids = jnp.sum(jnp.where(lane == j, idxm, 0), axis=1,
                      keepdims=True) - t * POOL_SIZE
        sel = (pool_iota == ids).astype(jnp.float32)
        slabs.append(lax.dot_general(
            sel, p[t],
            dimension_numbers=(((1,), (0,)), ((), ())),
            preferred_element_type=jnp.float32)[None])
    out_ref[...] = jnp.concatenate(slabs, axis=0)


def _tc_scatter(out_sc, p_t, idxb):
    return pl.pallas_call(
        _tc_scatter_body,
        grid=(_NSLAB_TC // PROMPT_LEN,),
        in_specs=[
            pl.BlockSpec(memory_space=pl.ANY),
            pl.BlockSpec((PROMPT_LEN, POOL_SIZE, EMBED_DIM),
                         lambda s: (0, 0, 0)),
            pl.BlockSpec((BATCH, _NSLAB_PAD), lambda s: (0, 0)),
        ],
        out_specs=pl.BlockSpec((PROMPT_LEN, BATCH, EMBED_DIM),
                               lambda s: (s, 0, 0)),
        out_shape=jax.ShapeDtypeStruct((_NSLAB, BATCH, EMBED_DIM),
                                       jnp.float32),
        input_output_aliases={0: 0},
    )(out_sc, p_t, idxb)


def kernel(input_embed, prompt):
    x_t = jnp.transpose(input_embed, (1, 0, 2))
    p_t = jnp.transpose(prompt, (1, 0, 2))
    idx, idxb = _topk_idx(x_t, p_t)
    table = p_t.reshape(PROMPT_LEN * POOL_SIZE, EMBED_DIM)
    out_sc = _sc_gather(table, idx)
    out_t = _tc_scatter(out_sc, p_t, idxb)
    return jnp.transpose(out_t, (1, 0, 2))
